# Initial kernel scaffold; baseline (speedup 1.0000x reference)
import jax
import jax.numpy as jnp
from jax.experimental import pallas as pl


def kernel(x):
    raise NotImplementedError("write your pallas kernel here")


# speedup vs baseline: 21.7367x; 21.7367x over previous
import functools

import jax
import jax.numpy as jnp
from jax import lax
from jax.experimental import pallas as pl
from jax.experimental.pallas import tpu as pltpu

B, N, C = 16, 2048, 256
K = 3
KM_ITERS = 10
BISECT_ITERS = 48


def _heatmap_body(x_ref, hm_ref):
    hm_ref[0, 0, :] = jnp.sum(x_ref[0], axis=-1) * (1.0 / C)


def _heatmap(x):
    return pl.pallas_call(
        _heatmap_body,
        grid=(B,),
        in_specs=[pl.BlockSpec((1, N, C), lambda b: (b, 0, 0))],
        out_specs=pl.BlockSpec((1, 1, N), lambda b: (b, 0, 0)),
        out_shape=jax.ShapeDtypeStruct((B, 1, N), jnp.float32),
    )(x)


def _order_stat(hm, k):
    lo = jnp.min(hm, axis=1, keepdims=True) - 1.0
    hi = jnp.max(hm, axis=1, keepdims=True)

    def body(_, carry):
        lo, hi = carry
        mid = 0.5 * (lo + hi)
        cnt = jnp.sum(jnp.where(hm <= mid, 1.0, 0.0), axis=1, keepdims=True)
        ge = cnt >= (k + 1)
        return jnp.where(ge, lo, mid), jnp.where(ge, mid, hi)

    lo, hi = lax.fori_loop(0, BISECT_ITERS, body, (lo, hi))
    return hi


def _cluster_body(hm_ref, lab_ref):
    hm = hm_ref[:, 0, :]
    c0 = jnp.min(hm, axis=1, keepdims=True)
    c2 = jnp.max(hm, axis=1, keepdims=True)
    c1 = 0.5 * (_order_stat(hm, N // 2 - 1) + _order_stat(hm, N // 2))
    total_s = jnp.sum(hm, axis=1, keepdims=True)

    def lloyd(_, carry):
        c0, c1, c2 = carry
        t01 = 0.5 * (c0 + c1)
        t12 = 0.5 * (c1 + c2)
        m1 = hm > t01
        m2 = hm > t12
        s0 = jnp.sum(jnp.where(m1, 0.0, hm), axis=1, keepdims=True)
        s2 = jnp.sum(jnp.where(m2, hm, 0.0), axis=1, keepdims=True)
        n0 = jnp.sum(jnp.where(m1, 0.0, 1.0), axis=1, keepdims=True)
        n2 = jnp.sum(jnp.where(m2, 1.0, 0.0), axis=1, keepdims=True)
        s1 = total_s - s0 - s2
        n1 = N - n0 - n2
        c0 = jnp.where(n0 > 0, s0 / jnp.maximum(n0, 1.0), c0)
        c1 = jnp.where(n1 > 0, s1 / jnp.maximum(n1, 1.0), c1)
        c2 = jnp.where(n2 > 0, s2 / jnp.maximum(n2, 1.0), c2)
        return c0, c1, c2

    c0, c1, c2 = lax.fori_loop(0, KM_ITERS, lloyd, (c0, c1, c2))

    t01 = 0.5 * (c0 + c1)
    t12 = 0.5 * (c1 + c2)
    lab = jnp.where(hm > t01, 1, 0) + jnp.where(hm > t12, 1, 0)

    a0 = (jnp.where(c1 > c0, 1, 0) + jnp.where(c2 > c0, 1, 0))
    a1 = (jnp.where(c0 > c1, 1, 0) + jnp.where(c2 > c1, 1, 0)
          + jnp.where(c0 == c1, 1, 0))
    a2 = (jnp.where(c0 > c2, 1, 0) + jnp.where(c1 > c2, 1, 0)
          + jnp.where(c0 == c2, 1, 0) + jnp.where(c1 == c2, 1, 0))
    adj = jnp.where(lab == 0, a0, jnp.where(lab == 1, a1, a2))
    lab_ref[:, 0, :] = adj


def _cluster_tc(hm):
    return pl.pallas_call(
        _cluster_body,
        in_specs=[pl.BlockSpec((B, 1, N), lambda: (0, 0, 0))],
        out_specs=pl.BlockSpec((B, 1, N), lambda: (0, 0, 0)),
        out_shape=jax.ShapeDtypeStruct((B, 1, N), jnp.int32),
    )(hm)


def _pool_body(x_ref, lab_ref, out_ref):
    lab = lab_ref[0, 0, :]
    oh = jnp.where(lab[None, :] == jax.lax.broadcasted_iota(jnp.int32, (K, N), 0),
                   1.0, 0.0)
    sums = jax.lax.dot(oh, x_ref[0], precision=jax.lax.Precision.HIGHEST,
                       preferred_element_type=jnp.float32)
    counts = jnp.sum(oh, axis=1, keepdims=True)
    out_ref[0] = sums / jnp.maximum(counts, 1.0)


def _pool(x, labels):
    return pl.pallas_call(
        _pool_body,
        grid=(B,),
        in_specs=[pl.BlockSpec((1, N, C), lambda b: (b, 0, 0)),
                  pl.BlockSpec((1, 1, N), lambda b: (b, 0, 0))],
        out_specs=pl.BlockSpec((1, K, C), lambda b: (b, 0, 0)),
        out_shape=jax.ShapeDtypeStruct((B, K, C), jnp.float32),
    )(x, labels)


@jax.jit
def kernel(x):
    hm = _heatmap(x)
    labels = _cluster_tc(hm)
    means = _pool(x, labels)
    return tuple(means[:, i, :] for i in range(K))
